# baseline (speedup 1.0000x reference)
import jax
import jax.numpy as jnp
from jax import lax
from jax.experimental import pallas as pl
from jax.experimental.pallas import tpu as pltpu
from jax.experimental.pallas import tpu_sc as plsc

B = 16384
F_CAT = 26
F_NUM = 13
V = 100000
D = 16
H = 32

_NC = 2
_NS = 16
_NW = _NC * _NS
_NR = F_CAT * D
_RPW = _NR // _NW
_QUART = B // 4


def _gather_body(tab_hbm, idx_hbm, out_hbm, row_v, idx_v, out_v, sem, osem):
    wid = lax.axis_index("s") * _NC + lax.axis_index("c")
    pending = [None, None]
    for j in range(_RPW):
        r = wid * _RPW + j
        f = r // D
        dr = pltpu.async_copy(tab_hbm.at[r], row_v, sem)
        if j == 0:
            pltpu.sync_copy(idx_hbm.at[f], idx_v)
        else:
            @pl.when(r % D == 0)
            def _load_idx(f=f):
                pltpu.sync_copy(idx_hbm.at[f], idx_v)
        dr.wait()
        for q in range(4):
            b = (j * 4 + q) % 2
            if pending[b] is not None:
                pending[b].wait()
            @plsc.parallel_loop(0, _QUART // 16, unroll=8)
            def _gather_iter(k, q=q, b=b):
                base = q * _QUART + k * 16
                vals = plsc.load_gather(row_v, [idx_v[pl.ds(base, 16)]])
                out_v[b, pl.ds(k * 16, 16)] = vals
            pending[b] = pltpu.async_copy(
                out_v.at[b], out_hbm.at[r, pl.ds(q * _QUART, _QUART)], osem)
    for p in pending:
        p.wait()


def _sc_gather(tab_rows, idx_cols):
    mesh = plsc.VectorSubcoreMesh(
        core_axis_name="c", subcore_axis_name="s",
        num_cores=_NC, num_subcores=_NS,
    )
    fn = pl.kernel(
        _gather_body,
        out_type=jax.ShapeDtypeStruct((_NR, B), jnp.float32),
        mesh=mesh,
        scratch_types=[
            pltpu.VMEM((V,), jnp.float32),
            pltpu.VMEM((B,), jnp.int32),
            pltpu.VMEM((2, _QUART), jnp.float32),
            pltpu.SemaphoreType.DMA,
            pltpu.SemaphoreType.DMA,
        ],
        compiler_params=pltpu.CompilerParams(
            use_tc_tiling_on_sc=True, needs_layout_passes=False,
        ),
    )
    return fn(tab_rows, idx_cols)


def _mlp_body(embT, numT, meanc, stdc, w1eT, w1nT, b1c, w2T, b2c, outT):
    nn = (numT[...] - meanc[...]) / stdc[...]
    x = jnp.dot(w1eT[...], embT[...], preferred_element_type=jnp.float32)
    x = x + jnp.dot(w1nT[...], nn, preferred_element_type=jnp.float32)
    x = jnp.maximum(x + b1c[...], 0.0)
    outT[...] = jnp.dot(w2T[...], x, preferred_element_type=jnp.float32) + b2c[...]


def _mlp(embT, numeric, norm_mean, norm_std, W1, b1, W2, b2):
    BT = 2048
    E = F_CAT * D
    outT = pl.pallas_call(
        _mlp_body,
        grid=(B // BT,),
        in_specs=[
            pl.BlockSpec((E, BT), lambda i: (0, i)),
            pl.BlockSpec((F_NUM, BT), lambda i: (0, i)),
            pl.BlockSpec((F_NUM, 1), lambda i: (0, 0)),
            pl.BlockSpec((F_NUM, 1), lambda i: (0, 0)),
            pl.BlockSpec((H, E), lambda i: (0, 0)),
            pl.BlockSpec((H, F_NUM), lambda i: (0, 0)),
            pl.BlockSpec((H, 1), lambda i: (0, 0)),
            pl.BlockSpec((1, H), lambda i: (0, 0)),
            pl.BlockSpec((1, 1), lambda i: (0, 0)),
        ],
        out_specs=pl.BlockSpec((1, BT), lambda i: (0, i)),
        out_shape=jax.ShapeDtypeStruct((1, B), jnp.float32),
    )(
        embT, jnp.transpose(numeric),
        norm_mean.reshape(F_NUM, 1), norm_std.reshape(F_NUM, 1),
        jnp.transpose(W1[:E]), jnp.transpose(W1[E:]),
        b1.reshape(H, 1), jnp.transpose(W2), b2.reshape(1, 1),
    )
    return outT.reshape(B, 1)


def kernel(cat_indices, numeric, tables, norm_mean, norm_std, W1, b1, W2, b2):
    tab_rows = jnp.transpose(tables, (0, 2, 1)).reshape(_NR, V)
    idx_cols = jnp.transpose(cat_indices).astype(jnp.int32)
    embT = _sc_gather(tab_rows, idx_cols)
    return _mlp(embT, numeric, norm_mean, norm_std, W1, b1, W2, b2)


# speedup vs baseline: 1.0245x; 1.0245x over previous
import jax
import jax.numpy as jnp
from jax import lax
from jax.experimental import pallas as pl
from jax.experimental.pallas import tpu as pltpu
from jax.experimental.pallas import tpu_sc as plsc

B = 16384
F_CAT = 26
F_NUM = 13
V = 100000
D = 16
H = 32

_NC = 2
_NS = 16
_NW = _NC * _NS
_NR = F_CAT * D
_RPW = _NR // _NW
_HALF = B // 2


def _gather_body(tab_hbm, idx_hbm, out_hbm, row_v, idx_v, out_v):
    wid = lax.axis_index("s") * _NC + lax.axis_index("c")
    for j in range(_RPW):
        r = wid * _RPW + j
        f = r // D
        if j == 0:
            pltpu.sync_copy(idx_hbm.at[f], idx_v)
        else:
            @pl.when(r % D == 0)
            def _load_idx(f=f):
                pltpu.sync_copy(idx_hbm.at[f], idx_v)
        pltpu.sync_copy(tab_hbm.at[r], row_v)
        for h in range(2):
            @plsc.parallel_loop(0, _HALF // 16, unroll=8)
            def _gather_iter(k, h=h):
                base = h * _HALF + k * 16
                vals = plsc.load_gather(row_v, [idx_v[pl.ds(base, 16)]])
                out_v[pl.ds(k * 16, 16)] = vals
            pltpu.sync_copy(out_v, out_hbm.at[r, pl.ds(h * _HALF, _HALF)])


def _sc_gather(tab_rows, idx_cols):
    mesh = plsc.VectorSubcoreMesh(
        core_axis_name="c", subcore_axis_name="s",
        num_cores=_NC, num_subcores=_NS,
    )
    fn = pl.kernel(
        _gather_body,
        out_type=jax.ShapeDtypeStruct((_NR, B), jnp.float32),
        mesh=mesh,
        scratch_types=[
            pltpu.VMEM((V,), jnp.float32),
            pltpu.VMEM((B,), jnp.int32),
            pltpu.VMEM((_HALF,), jnp.float32),
        ],
        compiler_params=pltpu.CompilerParams(
            use_tc_tiling_on_sc=True, needs_layout_passes=False,
        ),
    )
    return fn(tab_rows, idx_cols)


def _mlp_body(embT, numT, meanc, stdc, w1eT, w1nT, b1c, w2T, b2c, outT):
    nn = (numT[...] - meanc[...]) / stdc[...]
    x = jnp.dot(w1eT[...], embT[...], preferred_element_type=jnp.float32)
    x = x + jnp.dot(w1nT[...], nn, preferred_element_type=jnp.float32)
    x = jnp.maximum(x + b1c[...], 0.0)
    outT[...] = jnp.dot(w2T[...], x, preferred_element_type=jnp.float32) + b2c[...]


def _mlp(embT, numeric, norm_mean, norm_std, W1, b1, W2, b2):
    BT = 2048
    E = F_CAT * D
    outT = pl.pallas_call(
        _mlp_body,
        grid=(B // BT,),
        in_specs=[
            pl.BlockSpec((E, BT), lambda i: (0, i)),
            pl.BlockSpec((F_NUM, BT), lambda i: (0, i)),
            pl.BlockSpec((F_NUM, 1), lambda i: (0, 0)),
            pl.BlockSpec((F_NUM, 1), lambda i: (0, 0)),
            pl.BlockSpec((H, E), lambda i: (0, 0)),
            pl.BlockSpec((H, F_NUM), lambda i: (0, 0)),
            pl.BlockSpec((H, 1), lambda i: (0, 0)),
            pl.BlockSpec((1, H), lambda i: (0, 0)),
            pl.BlockSpec((1, 1), lambda i: (0, 0)),
        ],
        out_specs=pl.BlockSpec((1, BT), lambda i: (0, i)),
        out_shape=jax.ShapeDtypeStruct((1, B), jnp.float32),
    )(
        embT, jnp.transpose(numeric),
        norm_mean.reshape(F_NUM, 1), norm_std.reshape(F_NUM, 1),
        jnp.transpose(W1[:E]), jnp.transpose(W1[E:]),
        b1.reshape(H, 1), jnp.transpose(W2), b2.reshape(1, 1),
    )
    return outT.reshape(B, 1)


def kernel(cat_indices, numeric, tables, norm_mean, norm_std, W1, b1, W2, b2):
    tab_rows = jnp.transpose(tables, (0, 2, 1)).reshape(_NR, V)
    idx_cols = jnp.transpose(cat_indices).astype(jnp.int32)
    embT = _sc_gather(tab_rows, idx_cols)
    return _mlp(embT, numeric, norm_mean, norm_std, W1, b1, W2, b2)
